# Initial kernel scaffold; baseline (speedup 1.0000x reference)
import jax
import jax.numpy as jnp
from jax.experimental import pallas as pl


def kernel(x, edge_index, edge_attr, W_enc, b_enc, g_enc, be_enc, W_gcn0, b_gcn0, g_ln0, b_ln0, W_gcn1, b_gcn1, g_ln1, b_ln1, W_gcn2, b_gcn2, g_ln2, b_ln2, W_c1, b_c1, W_c2, b_c2, W_c3, b_c3):
    raise NotImplementedError("write your pallas kernel here")


# speedup vs baseline: 5.7238x; 5.7238x over previous
import functools

import jax
import jax.numpy as jnp
from jax import lax
from jax.experimental import pallas as pl
from jax.experimental.pallas import tpu as pltpu
from jax.experimental.pallas import tpu_sc as plsc

N_NODES = 10000
H = 128
NC = 2
NS = 16
NT = NC * NS
CH = 128
LN_EPS = 1e-5


def _mesh():
    return plsc.VectorSubcoreMesh(
        core_axis_name="c", subcore_axis_name="s", num_cores=NC, num_subcores=NS)


def _fori(n, body):
    lax.fori_loop(0, n, lambda i, c: (body(i), c)[1], 0)


def _sc_degree(dst_s, nacc, ept):
    nch = ept // CH
    rps = nacc // NS

    @functools.partial(
        pl.kernel,
        out_type=jax.ShapeDtypeStruct((NC, nacc, 16), jnp.float32),
        mesh=_mesh(),
        scratch_types=[
            pltpu.VMEM_SHARED((nacc, 16), jnp.float32),
            pltpu.VMEM((CH, 16), jnp.float32),
            pltpu.VMEM((CH, 16), jnp.float32),
            pltpu.VMEM((CH,), jnp.int32),
        ],
    )
    def k(dstr, out, acc, ones, zbuf, idst):
        c = lax.axis_index("c")
        s = lax.axis_index("s")
        ebase = (c * NS + s) * ept
        base = s * rps

        def init_row(r):
            ones[r, :] = jnp.full((16,), 1.0, jnp.float32)
            zbuf[r, :] = jnp.zeros((16,), jnp.float32)
        _fori(CH, init_row)

        def zero_chunk(i):
            pltpu.sync_copy(zbuf, acc.at[pl.ds(base + i * CH, CH)])
        _fori(rps // CH, zero_chunk)
        plsc.subcore_barrier()

        def step(j):
            pltpu.sync_copy(dstr.at[pl.ds(ebase + j * CH, CH)], idst)
            pltpu.sync_copy(ones, acc.at[idst], add=True)
        _fori(nch, step)
        plsc.subcore_barrier()

        def wb(i):
            sl = pl.ds(base + i * CH, CH)
            pltpu.sync_copy(acc.at[sl], out.at[c, sl])
        _fori(rps // CH, wb)

    return k(dst_s)


def _sc_scatter(table, src_p, dst_s, nacc, ept):
    nch = ept // CH
    rps = nacc // NS

    @functools.partial(
        pl.kernel,
        out_type=jax.ShapeDtypeStruct((NC, nacc, H), jnp.float32),
        mesh=_mesh(),
        scratch_types=[
            pltpu.VMEM_SHARED((nacc, H), jnp.float32),
            pltpu.VMEM((CH, H), jnp.float32),
            pltpu.VMEM((CH, H), jnp.float32),
            pltpu.VMEM((CH,), jnp.int32),
            pltpu.VMEM((CH,), jnp.int32),
            pltpu.SemaphoreType.DMA,
        ],
    )
    def k(tab, srcr, dstr, out, acc, rows, zbuf, isrc, idst, sem):
        c = lax.axis_index("c")
        s = lax.axis_index("s")
        ebase = (c * NS + s) * ept
        base = s * rps

        def zrow(r):
            for q in range(H // 16):
                zbuf[r, pl.ds(q * 16, 16)] = jnp.zeros((16,), jnp.float32)
        _fori(CH, zrow)

        def zero_chunk(i):
            pltpu.sync_copy(zbuf, acc.at[pl.ds(base + i * CH, CH)])
        _fori(rps // CH, zero_chunk)
        plsc.subcore_barrier()

        def step(j):
            off = ebase + j * CH
            pltpu.sync_copy(srcr.at[pl.ds(off, CH)], isrc)
            pltpu.sync_copy(dstr.at[pl.ds(off, CH)], idst)
            pltpu.async_copy(tab.at[isrc], rows, sem).wait()
            pltpu.sync_copy(rows, acc.at[idst], add=True)
        _fori(nch, step)
        plsc.subcore_barrier()

        def wb(i):
            sl = pl.ds(base + i * CH, CH)
            pltpu.sync_copy(acc.at[sl], out.at[c, sl])
        _fori(rps // CH, wb)

    return k(table, src_p, dst_s)


def _sc_pair_gather(A, B, src_p, dst_p, e_pad, ept):
    nch = ept // CH

    @functools.partial(
        pl.kernel,
        out_type=jax.ShapeDtypeStruct((e_pad, H), jnp.float32),
        mesh=_mesh(),
        scratch_types=[
            pltpu.VMEM((CH, H), jnp.float32),
            pltpu.VMEM((CH, H), jnp.float32),
            pltpu.VMEM((CH,), jnp.int32),
            pltpu.VMEM((CH,), jnp.int32),
            pltpu.SemaphoreType.DMA,
            pltpu.SemaphoreType.DMA,
        ],
    )
    def k(a_t, b_t, srcr, dstr, out, buf1, buf2, isrc, idst, sem1, sem2):
        c = lax.axis_index("c")
        s = lax.axis_index("s")
        ebase = (c * NS + s) * ept

        def step(j):
            off = ebase + j * CH
            pltpu.sync_copy(srcr.at[pl.ds(off, CH)], isrc)
            pltpu.sync_copy(dstr.at[pl.ds(off, CH)], idst)
            cp1 = pltpu.async_copy(a_t.at[isrc], buf1, sem1)
            cp2 = pltpu.async_copy(b_t.at[idst], buf2, sem2)
            cp1.wait()
            cp2.wait()

            def addrow(r):
                for q in range(H // 16):
                    sl = pl.ds(q * 16, 16)
                    buf1[r, sl] = buf1[r, sl] + buf2[r, sl]
            _fori(CH, addrow)
            pltpu.sync_copy(buf1, out.at[pl.ds(off, CH)])
        _fori(nch, step)

    return k(A, B, src_p, dst_p)


def _dis_block(degp):
    deg = degp[0, :, 0:1] + degp[1, :, 0:1] + 1.0
    return lax.rsqrt(deg)


def _ln_relu(h, g, b):
    m = jnp.mean(h, axis=-1, keepdims=True)
    v = jnp.mean((h - m) ** 2, axis=-1, keepdims=True)
    return jnp.maximum((h - m) * lax.rsqrt(v + LN_EPS) * g + b, 0.0)


def _tc_enc(x, WencT, b_enc, g_enc, be_enc, W0T, degp, bn):
    n = x.shape[0]
    grid = (n // bn,)
    row = lambda i: (i, 0)
    full = lambda i: (0, 0)

    def body(x_r, wt_r, be_r, g_r, bb_r, w0_r, dg_r, h_r, hs_r):
        h = jnp.dot(x_r[...], wt_r[...], preferred_element_type=jnp.float32)
        h = _ln_relu(h + be_r[...], g_r[...], bb_r[...])
        h_r[...] = h
        dis = _dis_block(dg_r)
        hs_r[...] = jnp.dot(h, w0_r[...], preferred_element_type=jnp.float32) * dis

    return pl.pallas_call(
        body,
        grid=grid,
        in_specs=[
            pl.BlockSpec((bn, H), row),
            pl.BlockSpec((H, H), full),
            pl.BlockSpec((1, H), full),
            pl.BlockSpec((1, H), full),
            pl.BlockSpec((1, H), full),
            pl.BlockSpec((H, H), full),
            pl.BlockSpec((NC, bn, 16), lambda i: (0, i, 0)),
        ],
        out_specs=[pl.BlockSpec((bn, H), row), pl.BlockSpec((bn, H), row)],
        out_shape=[jax.ShapeDtypeStruct((n, H), jnp.float32)] * 2,
    )(x, WencT, b_enc, g_enc, be_enc, W0T, degp)


def _tc_mid(parts, hs, h_prev, degp, b_l, g_l, bln_l, WnT, bn, last, W1sT=None,
            b_c1=None, W1dT=None):
    n = hs.shape[0]
    grid = (n // bn,)
    row = lambda i: (i, 0)
    full = lambda i: (0, 0)
    part_spec = pl.BlockSpec((NC, bn, H), lambda i: (0, i, 0))

    def combine(p_r, hs_r, hp_r, dg_r, b_r, g_r, bb_r):
        dis = _dis_block(dg_r)
        o = (p_r[0] + p_r[1] + hs_r[...]) * dis + b_r[...]
        return _ln_relu(o, g_r[...], bb_r[...]) + hp_r[...], dis

    if last:
        def body(p_r, hs_r, hp_r, dg_r, b_r, g_r, bb_r, w1s_r, bc1_r, w1d_r,
                 a_r, bt_r):
            hn, _ = combine(p_r, hs_r, hp_r, dg_r, b_r, g_r, bb_r)
            a_r[...] = jnp.dot(hn, w1s_r[...], preferred_element_type=jnp.float32) + bc1_r[...]
            bt_r[...] = jnp.dot(hn, w1d_r[...], preferred_element_type=jnp.float32)
        extra_in = [W1sT, b_c1, W1dT]
        extra_specs = [pl.BlockSpec((H, H), full), pl.BlockSpec((1, H), full),
                       pl.BlockSpec((H, H), full)]
    else:
        def body(p_r, hs_r, hp_r, dg_r, b_r, g_r, bb_r, wn_r, h_r, hs2_r):
            hn, dis = combine(p_r, hs_r, hp_r, dg_r, b_r, g_r, bb_r)
            h_r[...] = hn
            hs2_r[...] = jnp.dot(hn, wn_r[...], preferred_element_type=jnp.float32) * dis
        extra_in = [WnT]
        extra_specs = [pl.BlockSpec((H, H), full)]

    return pl.pallas_call(
        body,
        grid=grid,
        in_specs=[
            part_spec,
            pl.BlockSpec((bn, H), row),
            pl.BlockSpec((bn, H), row),
            pl.BlockSpec((NC, bn, 16), lambda i: (0, i, 0)),
            pl.BlockSpec((1, H), full),
            pl.BlockSpec((1, H), full),
            pl.BlockSpec((1, H), full),
        ] + extra_specs,
        out_specs=[pl.BlockSpec((bn, H), row), pl.BlockSpec((bn, H), row)],
        out_shape=[jax.ShapeDtypeStruct((n, H), jnp.float32)] * 2,
    )(parts, hs, h_prev, degp, b_l, g_l, bln_l, *extra_in)


def _tc_cls(P, ea, W1eT, W2T, b_c2, w3row, b_c3, be):
    e_pad = P.shape[0]
    grid = (e_pad // be,)
    row = lambda i: (i, 0)
    full = lambda i: (0, 0)

    def body(p_r, ea_r, w1e_r, w2_r, b2_r, w3_r, b3_r, o_r):
        o1 = p_r[...] + jnp.dot(ea_r[...], w1e_r[...],
                                preferred_element_type=jnp.float32)
        o1 = jnp.maximum(o1, 0.0)
        o2 = jnp.dot(o1, w2_r[...], preferred_element_type=jnp.float32) + b2_r[...]
        o2 = jnp.maximum(o2, 0.0)
        o_r[...] = jnp.sum(o2 * w3_r[...], axis=-1, keepdims=True) + b3_r[...]

    return pl.pallas_call(
        body,
        grid=grid,
        in_specs=[
            pl.BlockSpec((be, H), row),
            pl.BlockSpec((be, 16), row),
            pl.BlockSpec((16, H), full),
            pl.BlockSpec((H, H // 2), full),
            pl.BlockSpec((1, H // 2), full),
            pl.BlockSpec((1, H // 2), full),
            pl.BlockSpec((1, 1), full),
        ],
        out_specs=pl.BlockSpec((be, 1), row),
        out_shape=jax.ShapeDtypeStruct((e_pad, 1), jnp.float32),
    )(P, ea, W1eT, W2T, b_c2, w3row, b_c3)


def kernel(x, edge_index, edge_attr, W_enc, b_enc, g_enc, be_enc,
           W_gcn0, b_gcn0, g_ln0, b_ln0,
           W_gcn1, b_gcn1, g_ln1, b_ln1,
           W_gcn2, b_gcn2, g_ln2, b_ln2,
           W_c1, b_c1, W_c2, b_c2, W_c3, b_c3):
    n, h = x.shape
    e = edge_index.shape[1]
    ept = -(-e // (NT * CH)) * CH
    e_pad = ept * NT
    nacc = -(-n // (NS * CH)) * NS * CH

    src = edge_index[0]
    dst = edge_index[1]
    pad = e_pad - e
    src_p = jnp.pad(src, (0, pad))
    dst_p = jnp.pad(dst, (0, pad))
    dst_s = jnp.pad(dst, (0, pad), constant_values=n)
    ea_p = jnp.pad(edge_attr, ((0, pad), (0, 0)))

    r2 = lambda v: v.reshape(1, -1)
    WencT = W_enc.T
    W0T, W1T, W2T_g = W_gcn0.T, W_gcn1.T, W_gcn2.T
    W1sT = W_c1[:, :h].T
    W1eT = W_c1[:, h:h + 16].T
    W1dT = W_c1[:, h + 16:].T
    Wc2T = W_c2.T

    bn = 1000

    degp = _sc_degree(dst_s, nacc, ept)

    h0, hs0 = _tc_enc(x, WencT, r2(b_enc), r2(g_enc), r2(be_enc), W0T, degp, bn)

    parts0 = _sc_scatter(hs0, src_p, dst_s, nacc, ept)
    h1, hs1 = _tc_mid(parts0, hs0, h0, degp, r2(b_gcn0), r2(g_ln0), r2(b_ln0),
                      W1T, bn, last=False)
    parts1 = _sc_scatter(hs1, src_p, dst_s, nacc, ept)
    h2, hs2 = _tc_mid(parts1, hs1, h1, degp, r2(b_gcn1), r2(g_ln1), r2(b_ln1),
                      W2T_g, bn, last=False)
    parts2 = _sc_scatter(hs2, src_p, dst_s, nacc, ept)
    A, B = _tc_mid(parts2, hs2, h2, degp, r2(b_gcn2), r2(g_ln2), r2(b_ln2),
                   None, bn, last=True, W1sT=W1sT, b_c1=r2(b_c1), W1dT=W1dT)

    P = _sc_pair_gather(A, B, src_p, dst_p, e_pad, ept)

    out = _tc_cls(P, ea_p, W1eT, Wc2T, r2(b_c2), W_c3, r2(b_c3), be=2048)
    return out[:e]
